# baseline (speedup 1.0000x reference)
import functools

import jax
import jax.numpy as jnp
from jax import lax
from jax.experimental import pallas as pl
from jax.experimental.pallas import tpu as pltpu
from jax.experimental.pallas import tpu_sc as plsc

_B = 16384
_D = 64


@functools.cache
def _build_gather(num_cores: int, num_subcores: int):
    nw = num_cores * num_subcores
    b_per_w = _B // nw
    mesh = plsc.VectorSubcoreMesh(core_axis_name="c", subcore_axis_name="s")

    @functools.partial(
        pl.kernel,
        mesh=mesh,
        out_type=jax.ShapeDtypeStruct((_B, 2 * _D), jnp.float32),
        scratch_types=[
            pltpu.VMEM((b_per_w,), jnp.int32),
            pltpu.VMEM((b_per_w,), jnp.int32),
            pltpu.VMEM((b_per_w, 2 * _D), jnp.float32),
            pltpu.SemaphoreType.DMA,
        ],
    )
    def gather_kernel(table_hbm, idx_hbm, out_hbm, idx_v, m_v, rows_v, sem):
        wid = lax.axis_index("s") * num_cores + lax.axis_index("c")
        base = wid * b_per_w
        pltpu.sync_copy(idx_hbm.at[pl.ds(base, b_per_w)], idx_v)

        def compute_m(i, carry):
            v = idx_v[pl.ds(i * 16, 16)]
            m_v[pl.ds(i * 16, 16)] = lax.shift_right_logical(v, 1)
            return carry
        lax.fori_loop(0, b_per_w // 16, compute_m, 0)
        pltpu.async_copy(table_hbm.at[m_v], rows_v, sem).wait()

        def select_half(g, carry):
            vidx = idx_v[pl.ds(g * 16, 16)]
            for l in range(16):
                k = g * 16 + l
                h = lax.mul(lax.bitwise_and(vidx[l], 1), _D)
                for j in range(_D // 16):
                    rows_v[k, pl.ds(j * 16, 16)] = rows_v[k, pl.ds(h + j * 16, 16)]
            return carry
        lax.fori_loop(0, b_per_w // 16, select_half, 0)
        pltpu.sync_copy(rows_v, out_hbm.at[pl.ds(base, b_per_w)])

    return gather_kernel


def kernel(W, indices):
    idx = indices[:, 0].astype(jnp.int32)
    Wpair = W.reshape(50000, 2 * _D)
    info = plsc.get_sparse_core_info()
    gather = _build_gather(info.num_cores, info.num_subcores)
    out_p = gather(Wpair, idx)
    return out_p[:, :_D]
